# Initial kernel scaffold; baseline (speedup 1.0000x reference)
import jax
import jax.numpy as jnp
from jax.experimental import pallas as pl


def kernel(vert1, vert2, edge_points):
    raise NotImplementedError("write your pallas kernel here")


# speedup vs baseline: 14.4219x; 14.4219x over previous
import functools

import jax
import jax.numpy as jnp
from jax import lax
from jax.experimental import pallas as pl
from jax.experimental.pallas import tpu as pltpu
from jax.experimental.pallas import tpu_sc as plsc

NC = 2
NS = 16
NW = NC * NS
L = 16
CH = 512
SUB = 128
DPAD = 16


def _colv(c):
    return jnp.full((L,), c, dtype=jnp.int32)


@functools.lru_cache(maxsize=None)
def _build_sc(ni, be_pad, n_rows):
    mesh = plsc.VectorSubcoreMesh(core_axis_name="c", subcore_axis_name="s")

    def body(ep_hbm, table_hbm, out_hbm, idx_v, ra, rb, rc, rd, out_v, sem):
        wid = lax.axis_index("s") * NC + lax.axis_index("c")
        w_base = wid * (ni * CH)
        rbufs = (ra, rb, rc, rd)

        def chunk(i, carry):
            base = w_base + i * CH
            for v in range(4):
                pltpu.sync_copy(ep_hbm.at[v, pl.ds(base, CH)], idx_v.at[v])
            copies = []
            for v in range(4):
                for s in range(CH // SUB):
                    copies.append(pltpu.async_copy(
                        table_hbm.at[idx_v.at[v, pl.ds(s * SUB, SUB)]],
                        rbufs[v].at[pl.ds(s * SUB, SUB), :], sem))
            for cpy in copies:
                cpy.wait()
            for g in range(CH // L):
                ridx = lax.iota(jnp.int32, L) + (g * L)
                av = [plsc.load_gather(ra, [ridx, _colv(c)]) for c in range(6)]
                bv = [plsc.load_gather(rb, [ridx, _colv(c)]) for c in range(6)]
                cv = [plsc.load_gather(rc, [ridx, _colv(c)]) for c in range(6)]
                dv = [plsc.load_gather(rd, [ridx, _colv(c)]) for c in range(6)]
                sl = pl.ds(g * L, L)
                for t in range(2):
                    o = 3 * t
                    e0 = bv[o] - av[o]
                    e1 = bv[o + 1] - av[o + 1]
                    e2 = bv[o + 2] - av[o + 2]
                    f0 = cv[o] - av[o]
                    f1 = cv[o + 1] - av[o + 1]
                    f2 = cv[o + 2] - av[o + 2]
                    g0 = dv[o] - av[o]
                    g1 = dv[o + 1] - av[o + 1]
                    g2 = dv[o + 2] - av[o + 2]
                    n10 = e1 * f2 - e2 * f1
                    n11 = e2 * f0 - e0 * f2
                    n12 = e0 * f1 - e1 * f0
                    n20 = e1 * g2 - e2 * g1
                    n21 = e2 * g0 - e0 * g2
                    n22 = e0 * g1 - e1 * g0
                    out_v[3 * t + 0, sl] = n10 * n20 + n11 * n21 + n12 * n22
                    out_v[3 * t + 1, sl] = n10 * n10 + n11 * n11 + n12 * n12
                    out_v[3 * t + 2, sl] = n20 * n20 + n21 * n21 + n22 * n22
            for q in range(6):
                pltpu.sync_copy(out_v.at[q], out_hbm.at[q, pl.ds(base, CH)])
            return carry

        lax.fori_loop(0, ni, chunk, 0)

    return pl.kernel(
        body,
        out_type=jax.ShapeDtypeStruct((6, be_pad), jnp.float32),
        mesh=mesh,
        compiler_params=pltpu.CompilerParams(
            needs_layout_passes=False, use_tc_tiling_on_sc=False),
        scratch_types=[
            pltpu.VMEM((4, CH), jnp.int32),
            pltpu.VMEM((CH, DPAD), jnp.float32),
            pltpu.VMEM((CH, DPAD), jnp.float32),
            pltpu.VMEM((CH, DPAD), jnp.float32),
            pltpu.VMEM((CH, DPAD), jnp.float32),
            pltpu.VMEM((6, CH), jnp.float32),
            pltpu.SemaphoreType.DMA,
        ],
    )


@functools.lru_cache(maxsize=None)
def _build_tc(be_pad, nblk, denom):
    grid = (be_pad // nblk,)

    def body(rec_ref, out_ref):
        i = pl.program_id(0)

        @pl.when(i == 0)
        def _init():
            out_ref[0, 0] = 0.0

        d1 = rec_ref[0, :]
        a1 = rec_ref[1, :]
        b1 = rec_ref[2, :]
        d2 = rec_ref[3, :]
        a2 = rec_ref[4, :]
        b2 = rec_ref[5, :]
        eps = 1e-8
        c1 = d1 / ((jnp.sqrt(a1) + eps) * (jnp.sqrt(b1) + eps))
        c2 = d2 / ((jnp.sqrt(a2) + eps) * (jnp.sqrt(b2) + eps))
        lo, hi = -1.0 + 1e-6, 1.0 - 1e-6
        c1 = jnp.clip(c1, lo, hi)
        c2 = jnp.clip(c2, lo, hi)

        def _acos(x):
            return lax.atan2(jnp.sqrt((1.0 - x) * (1.0 + x)), x)

        t = _acos(c2) - _acos(c1)
        out_ref[0, 0] += jnp.sum(t * t)

        @pl.when(i == grid[0] - 1)
        def _fin():
            out_ref[0, 0] = out_ref[0, 0] / denom

    return pl.pallas_call(
        body,
        out_shape=jax.ShapeDtypeStruct((1, 1), jnp.float32),
        grid=grid,
        in_specs=[pl.BlockSpec((6, nblk), lambda i: (0, i))],
        out_specs=pl.BlockSpec((1, 1), lambda i: (0, 0),
                               memory_space=pltpu.SMEM),
    )


def kernel(vert1, vert2, edge_points):
    B, N, _ = vert1.shape
    E = edge_points.shape[1]
    BE = B * E
    ni = -(-BE // (NW * CH))
    be_pad = NW * CH * ni

    table = jnp.pad(
        jnp.concatenate([vert1, vert2], axis=-1).reshape(B * N, 6),
        ((0, 0), (0, DPAD - 6)))
    ep = (edge_points.astype(jnp.int32)
          + (jnp.arange(B, dtype=jnp.int32) * N)[:, None, None])
    ep = ep.transpose(2, 0, 1).reshape(4, BE)
    ep = jnp.pad(ep, ((0, 0), (0, be_pad - BE)))

    rec = _build_sc(ni, be_pad, B * N)(ep, table)
    res = _build_tc(be_pad, 16384, float(BE))(rec)
    return res[0, 0]


# speedup vs baseline: 26.7157x; 1.8524x over previous
import functools

import jax
import jax.numpy as jnp
import numpy as np
from jax import lax
from jax.experimental import pallas as pl
from jax.experimental.pallas import tpu as pltpu
from jax.experimental.pallas import tpu_sc as plsc

NC = 2
NS = 16
NW = NC * NS
L = 16
CH = 512
SUB = 128
DPAD = 8

_ACOS_POLY = (1.5707963050, -0.2145988016, 0.0889789874, -0.0501743046,
              0.0308918810, -0.0170881256, 0.0066700901, -0.0012624911)


def _f32(x):
    return jnp.float32(x)


def _rsqrt_nr(u):
    i = plsc.bitcast(u, jnp.int32)
    i = jnp.int32(0x5F3759DF) - lax.shift_right_logical(i, 1)
    y = plsc.bitcast(i, jnp.float32)
    for _ in range(3):
        y = y * (_f32(1.5) - _f32(0.5) * u * y * y)
    return y


def _acos(x):
    ax = jnp.abs(x)
    u = _f32(1.0) - ax
    s = u * _rsqrt_nr(u)
    p = jnp.full((L,), _ACOS_POLY[7], dtype=jnp.float32)
    for c in _ACOS_POLY[6::-1]:
        p = p * ax + _f32(c)
    a = s * p
    return jnp.where(x >= _f32(0.0), a, _f32(np.pi) - a)


def _edge_cos(av, bv, cv, dv, o):
    e0 = bv[o] - av[o]
    e1 = bv[o + 1] - av[o + 1]
    e2 = bv[o + 2] - av[o + 2]
    f0 = cv[o] - av[o]
    f1 = cv[o + 1] - av[o + 1]
    f2 = cv[o + 2] - av[o + 2]
    g0 = dv[o] - av[o]
    g1 = dv[o + 1] - av[o + 1]
    g2 = dv[o + 2] - av[o + 2]
    n10 = e1 * f2 - e2 * f1
    n11 = e2 * f0 - e0 * f2
    n12 = e0 * f1 - e1 * f0
    n20 = e1 * g2 - e2 * g1
    n21 = e2 * g0 - e0 * g2
    n22 = e0 * g1 - e1 * g0
    dot = n10 * n20 + n11 * n21 + n12 * n22
    s1 = n10 * n10 + n11 * n11 + n12 * n12
    s2 = n20 * n20 + n21 * n21 + n22 * n22
    eps = _f32(1e-8)
    den = (s1 * _rsqrt_nr(s1) + eps) * (s2 * _rsqrt_nr(s2) + eps)
    c = dot / den
    return jnp.clip(c, _f32(-1.0 + 1e-6), _f32(1.0 - 1e-6))


@functools.lru_cache(maxsize=None)
def _build_sc(ni, be_pad):
    mesh = plsc.VectorSubcoreMesh(core_axis_name="c", subcore_axis_name="s")
    nsub = CH // SUB

    def gathers(table_hbm, idx, rows, sem):
        handles = []
        for v in range(4):
            for s in range(nsub):
                handles.append(pltpu.async_copy(
                    table_hbm.at[idx.at[v, pl.ds(s * SUB, SUB)]],
                    rows.at[pl.ds(v * CH + s * SUB, SUB), :], sem))
        return handles

    def drain_gathers(table_hbm, idx, rows, sem):
        for v in range(4):
            for s in range(nsub):
                pltpu.make_async_copy(
                    table_hbm.at[idx.at[v, pl.ds(s * SUB, SUB)]],
                    rows.at[pl.ds(v * CH + s * SUB, SUB), :], sem).wait()

    def load_idx(ep_hbm, base, idx, sem):
        return pltpu.async_copy(ep_hbm.at[:, pl.ds(base, CH)], idx, sem)

    def compute(rows, acc):
        def group(g, acc):
            av, bv, cv, dv = [
                [plsc.load_gather(
                    rows,
                    [lax.iota(jnp.int32, L) + (g * L + v * CH),
                     jnp.full((L,), c, dtype=jnp.int32)])
                 for c in range(6)]
                for v in range(4)]
            c1 = _edge_cos(av, bv, cv, dv, 0)
            c2 = _edge_cos(av, bv, cv, dv, 3)
            t = _acos(c2) - _acos(c1)
            return acc + t * t

        return lax.fori_loop(0, CH // L, group, acc)

    def body(ep_hbm, table_hbm, out_hbm,
             idx_a, idx_b, rows_a, rows_b, accv,
             sem_a, sem_b, sem_ia, sem_ib, sem_out):
        wid = lax.axis_index("s") * NC + lax.axis_index("c")
        w_base = wid * (ni * CH)

        pltpu.sync_copy(ep_hbm.at[:, pl.ds(w_base, CH)], idx_a)
        gathers(table_hbm, idx_a, rows_a, sem_a)
        load_idx(ep_hbm, w_base + CH, idx_b, sem_ib)

        def pair(j, acc):
            c0 = 2 * j
            base0 = w_base + c0 * CH
            pltpu.make_async_copy(
                ep_hbm.at[:, pl.ds(base0 + CH, CH)], idx_b, sem_ib).wait()
            gathers(table_hbm, idx_b, rows_b, sem_b)
            drain_gathers(table_hbm, idx_a, rows_a, sem_a)

            @pl.when(c0 + 2 < ni)
            def _prefetch_a():
                load_idx(ep_hbm, base0 + 2 * CH, idx_a, sem_ia)

            acc = compute(rows_a, acc)

            @pl.when(c0 + 2 < ni)
            def _launch_a():
                pltpu.make_async_copy(
                    ep_hbm.at[:, pl.ds(base0 + 2 * CH, CH)], idx_a,
                    sem_ia).wait()
                gathers(table_hbm, idx_a, rows_a, sem_a)

            drain_gathers(table_hbm, idx_b, rows_b, sem_b)

            @pl.when(c0 + 3 < ni)
            def _prefetch_b():
                load_idx(ep_hbm, base0 + 3 * CH, idx_b, sem_ib)

            return compute(rows_b, acc)

        acc = lax.fori_loop(0, ni // 2, pair, jnp.zeros((L,), jnp.float32))
        accv[...] = acc
        pltpu.sync_copy(accv, out_hbm.at[wid])

    return pl.kernel(
        body,
        out_type=jax.ShapeDtypeStruct((NW, L), jnp.float32),
        mesh=mesh,
        compiler_params=pltpu.CompilerParams(
            needs_layout_passes=False, use_tc_tiling_on_sc=False),
        scratch_types=[
            pltpu.VMEM((4, CH), jnp.int32),
            pltpu.VMEM((4, CH), jnp.int32),
            pltpu.VMEM((4 * CH, DPAD), jnp.float32),
            pltpu.VMEM((4 * CH, DPAD), jnp.float32),
            pltpu.VMEM((L,), jnp.float32),
            pltpu.SemaphoreType.DMA,
            pltpu.SemaphoreType.DMA,
            pltpu.SemaphoreType.DMA,
            pltpu.SemaphoreType.DMA,
            pltpu.SemaphoreType.DMA,
        ],
    )


@functools.lru_cache(maxsize=None)
def _build_tc(denom):
    def body(part_ref, out_ref):
        out_ref[0, 0] = jnp.sum(part_ref[...]) * _f32(1.0 / denom)

    return pl.pallas_call(
        body,
        out_shape=jax.ShapeDtypeStruct((1, 1), jnp.float32),
        out_specs=pl.BlockSpec(memory_space=pltpu.SMEM),
    )


def kernel(vert1, vert2, edge_points):
    B, N, _ = vert1.shape
    E = edge_points.shape[1]
    BE = B * E
    ni = -(-BE // (NW * CH))
    ni += ni % 2
    be_pad = NW * CH * ni

    table = jnp.pad(
        jnp.concatenate([vert1, vert2], axis=-1).reshape(B * N, 6),
        ((0, 0), (0, DPAD - 6)))
    ep = (edge_points.astype(jnp.int32)
          + (jnp.arange(B, dtype=jnp.int32) * N)[:, None, None])
    ep = ep.transpose(2, 0, 1).reshape(4, BE)
    ep = jnp.pad(ep, ((0, 0), (0, be_pad - BE)))

    part = _build_sc(ni, be_pad)(ep, table)
    res = _build_tc(float(BE))(part)
    return res[0, 0]
